# baseline (speedup 1.0000x reference)
import jax
import jax.numpy as jnp
from jax.experimental import pallas as pl
from jax.experimental.pallas import tpu as pltpu

SEQ_LEN = 2048
BATCH = 8
NEG = -1000000000.0
RB = 256


def _body(n_ref, x_ref, o_ref):
    b = pl.program_id(0)
    rb = pl.program_id(1)
    nb = n_ref[b]
    rows = jax.lax.broadcasted_iota(jnp.int32, (RB, SEQ_LEN), 0) + rb * RB
    cols = jax.lax.broadcasted_iota(jnp.int32, (RB, SEQ_LEN), 1)
    valid = (rows < nb) & (cols < nb)
    o_ref[0] = jnp.where(valid, x_ref[0], jnp.float32(NEG))


def kernel(x, n):
    B, S, _ = x.shape
    grid = (B, S // RB)
    return pl.pallas_call(
        _body,
        grid=grid,
        in_specs=[
            pl.BlockSpec(memory_space=pltpu.SMEM),
            pl.BlockSpec((1, RB, S), lambda b, r: (b, r, 0)),
        ],
        out_specs=pl.BlockSpec((1, RB, S), lambda b, r: (b, r, 0)),
        out_shape=jax.ShapeDtypeStruct((B, S, S), jnp.float32),
    )(n, x)
